# baseline (speedup 1.0000x reference)
import functools

import jax
import jax.numpy as jnp
from jax import lax
from jax.experimental import pallas as pl
from jax.experimental.pallas import tpu as pltpu
from jax.experimental.pallas import tpu_sc as plsc

B = 4096
H = 50
D = 128
TOTAL = B * H

NC = 2
NS = 16
NW = NC * NS
B_PER_W = TOTAL // NW
CHUNK = 320
NCHUNK = B_PER_W // CHUNK

_mesh = plsc.VectorSubcoreMesh(core_axis_name="c", subcore_axis_name="s")


@functools.partial(
    pl.kernel,
    mesh=_mesh,
    out_type=jax.ShapeDtypeStruct((TOTAL, D), jnp.float32),
    scratch_types=[
        pltpu.VMEM((B_PER_W,), jnp.int32),
        pltpu.VMEM((CHUNK, D), jnp.float32),
        pltpu.VMEM((CHUNK, D), jnp.float32),
        pltpu.VMEM((CHUNK, D), jnp.float32),
        pltpu.SemaphoreType.DMA,
        pltpu.SemaphoreType.DMA,
        pltpu.SemaphoreType.DMA,
    ],
)
def _gather_kernel(idx_hbm, table_hbm, out_hbm,
                   idx_v, rows_a, rows_b, rows_c, sem_a, sem_b, sem_c):
    bufs = (rows_a, rows_b, rows_c)
    sems = (sem_a, sem_b, sem_c)
    wid = lax.axis_index("s") * NC + lax.axis_index("c")
    base = wid * B_PER_W
    pltpu.sync_copy(idx_hbm.at[pl.ds(base, B_PER_W)], idx_v)

    def start(j, b):
        pltpu.async_copy(table_hbm.at[idx_v.at[pl.ds(j * CHUNK, CHUNK)]],
                         bufs[b], sems[b])

    def finish(j, b):
        pltpu.make_async_copy(table_hbm.at[idx_v.at[pl.ds(j * CHUNK, CHUNK)]],
                              bufs[b], sems[b]).wait()
        pltpu.sync_copy(bufs[b], out_hbm.at[pl.ds(base + j * CHUNK, CHUNK)])

    for b in range(3):
        start(b, b)

    def body(t, carry):
        j = 3 * t
        for b in range(3):
            finish(j + b, b)
            start(j + b + 3, b)
        return carry

    lax.fori_loop(0, 5, body, 0)
    for j in range(15, NCHUNK):
        finish(j, j % 3)
        if j + 3 < NCHUNK:
            start(j + 3, j % 3)


def kernel(input_indices, token_embedding_table):
    idx = input_indices.T.reshape(-1).astype(jnp.int32)
    out = _gather_kernel(idx, token_embedding_table)
    return out.reshape(H, B, D).transpose(1, 0, 2)


# speedup vs baseline: 1.0094x; 1.0094x over previous
import functools

import jax
import jax.numpy as jnp
from jax import lax
from jax.experimental import pallas as pl
from jax.experimental.pallas import tpu as pltpu
from jax.experimental.pallas import tpu_sc as plsc

B = 4096
H = 50
D = 128
TOTAL = B * H

NC = 2
NS = 16
NW = NC * NS
B_PER_W = TOTAL // NW
CHUNK = 400
NCHUNK = B_PER_W // CHUNK

_mesh = plsc.VectorSubcoreMesh(core_axis_name="c", subcore_axis_name="s")


@functools.partial(
    pl.kernel,
    mesh=_mesh,
    out_type=jax.ShapeDtypeStruct((TOTAL, D), jnp.float32),
    scratch_types=[
        pltpu.VMEM((B_PER_W,), jnp.int32),
        pltpu.VMEM((CHUNK, D), jnp.float32),
        pltpu.VMEM((CHUNK, D), jnp.float32),
        pltpu.SemaphoreType.DMA,
        pltpu.SemaphoreType.DMA,
    ],
)
def _gather_kernel(idx_hbm, table_hbm, out_hbm,
                   idx_v, rows_a, rows_b, sem_a, sem_b):
    wid = lax.axis_index("s") * NC + lax.axis_index("c")
    base = wid * B_PER_W
    pltpu.sync_copy(idx_hbm.at[pl.ds(base, B_PER_W)], idx_v)

    def start(j, rows_v, sem):
        pltpu.async_copy(table_hbm.at[idx_v.at[pl.ds(j * CHUNK, CHUNK)]],
                         rows_v, sem)

    def finish(j, rows_v, sem):
        pltpu.make_async_copy(table_hbm.at[idx_v.at[pl.ds(j * CHUNK, CHUNK)]],
                              rows_v, sem).wait()
        pltpu.sync_copy(rows_v, out_hbm.at[pl.ds(base + j * CHUNK, CHUNK)])

    start(0, rows_a, sem_a)
    start(1, rows_b, sem_b)

    def body(t, carry):
        j = 2 * t
        finish(j, rows_a, sem_a)
        start(j + 2, rows_a, sem_a)
        finish(j + 1, rows_b, sem_b)
        start(j + 3, rows_b, sem_b)
        return carry

    lax.fori_loop(0, NCHUNK // 2 - 1, body, 0)
    finish(NCHUNK - 2, rows_a, sem_a)
    finish(NCHUNK - 1, rows_b, sem_b)


def kernel(input_indices, token_embedding_table):
    idx = input_indices.T.reshape(-1).astype(jnp.int32)
    out = _gather_kernel(idx, token_embedding_table)
    return out.reshape(H, B, D).transpose(1, 0, 2)
